# baseline (speedup 1.0000x reference)
import jax
import jax.numpy as jnp
from jax import lax
from jax.experimental import pallas as pl
from jax.experimental.pallas import tpu as pltpu
from jax.experimental.pallas import tpu_sc as plsc

TEMP = 0.07
N = 2048
C = 64
CP = 128
K = 4096
LOG2_HW = 14
LOG2_CHW = 20
NUM_WORKERS = 32
S_PER_W = N // NUM_WORKERS
LANES = 16


def _sc_gather_body(emb0_hbm, emb1_hbm, perm_hbm, out0_hbm, out1_hbm,
                    perm_v, idx_v, rows0_v, rows1_v, sem0, sem1):
    wid = lax.axis_index("s") * 2 + lax.axis_index("c")
    s_base = wid * S_PER_W
    pltpu.sync_copy(perm_hbm.at[pl.ds(s_base, S_PER_W)], perm_v)

    lane = lax.iota(jnp.int32, 16)
    bases = []
    pos0 = []
    for g in range(S_PER_W // LANES):
        p = perm_v[pl.ds(g * LANES, LANES)]
        b = lax.shift_right_logical(p, LOG2_HW)
        hw = lax.bitwise_and(p, (1 << LOG2_HW) - 1)
        bases.append(lax.shift_left(b, LOG2_CHW) + hw)
        pos0.append((g * LANES + lane) * C)

    def build(c, carry):
        cofs = lax.shift_left(c, LOG2_HW)
        for g in range(S_PER_W // LANES):
            pos = pos0[g] + c
            plsc.store_scatter(idx_v, [pos], bases[g] + cofs)
        return carry

    lax.fori_loop(0, C, build, 0)

    def fire(s, carry):
        pltpu.make_async_copy(
            emb0_hbm.at[idx_v.at[pl.ds(s * C, C)]],
            rows0_v.at[pl.ds(s * CP, C)], sem0).start()
        pltpu.make_async_copy(
            emb1_hbm.at[idx_v.at[pl.ds(s * C, C)]],
            rows1_v.at[pl.ds(s * CP, C)], sem1).start()
        return carry

    lax.fori_loop(0, S_PER_W, fire, 0)

    pltpu.make_async_copy(
        emb0_hbm.at[pl.ds(0, S_PER_W * C)],
        rows0_v.at[pl.ds(0, S_PER_W * C)], sem0).wait()
    pltpu.make_async_copy(
        emb1_hbm.at[pl.ds(0, S_PER_W * C)],
        rows1_v.at[pl.ds(0, S_PER_W * C)], sem1).wait()

    pltpu.sync_copy(rows0_v, out0_hbm.at[pl.ds(s_base * CP, S_PER_W * CP)])
    pltpu.sync_copy(rows1_v, out1_hbm.at[pl.ds(s_base * CP, S_PER_W * CP)])


def _sc_gather(emb0_flat, emb1_flat, perm):
    fn = pl.kernel(
        _sc_gather_body,
        mesh=plsc.VectorSubcoreMesh(core_axis_name="c", subcore_axis_name="s"),
        compiler_params=pltpu.CompilerParams(needs_layout_passes=False,
                                             use_tc_tiling_on_sc=False,
                                             skip_device_barrier=True),
        out_type=[jax.ShapeDtypeStruct((N * CP,), jnp.float32),
                  jax.ShapeDtypeStruct((N * CP,), jnp.float32)],
        scratch_types=[
            pltpu.VMEM((S_PER_W,), jnp.int32),
            pltpu.VMEM((S_PER_W * C,), jnp.int32),
            pltpu.VMEM((S_PER_W * CP,), jnp.float32),
            pltpu.VMEM((S_PER_W * CP,), jnp.float32),
            pltpu.SemaphoreType.DMA,
            pltpu.SemaphoreType.DMA,
        ],
    )
    return fn(emb0_flat, emb1_flat, perm)


BLK = 512


def _dense_body(q_ref, k_ref, queue_ref, ones_ref, out_ref):
    i = pl.program_id(0)
    good = lax.broadcasted_iota(jnp.int32, (BLK, CP), 1) < C
    q = jnp.where(good, q_ref[...], 0.0)
    kk = jnp.where(good, k_ref[...], 0.0)
    qn = q / (jnp.sqrt(jnp.sum(q * q, axis=1, keepdims=True)) + 1e-6)
    kn = kk / (jnp.sqrt(jnp.sum(kk * kk, axis=1, keepdims=True)) + 1e-6)
    z = jnp.sum(qn * kn, axis=1, keepdims=True) * (1.0 / TEMP)
    qs = (qn * (1.0 / TEMP)).astype(jnp.bfloat16)
    lneg = lax.dot_general(qs, queue_ref[...], (((1,), (1,)), ((), ())),
                           preferred_element_type=jnp.float32)
    el = jnp.exp(lneg).astype(jnp.bfloat16)
    sneg = lax.dot_general(el, ones_ref[...], (((1,), (0,)), ((), ())),
                           preferred_element_type=jnp.float32)[:, :1]
    s = jnp.exp(z) + sneg
    lse = jnp.log(s)
    partial = jnp.sum(lse - z) * (1.0 / N)

    @pl.when(i == 0)
    def _():
        out_ref[0, 0] = 0.0

    out_ref[0, 0] += partial


def _dense_loss(v0, v1, queue_pad, ones_pad):
    return pl.pallas_call(
        _dense_body,
        grid=(N // BLK,),
        in_specs=[
            pl.BlockSpec((BLK, CP), lambda i: (i, 0)),
            pl.BlockSpec((BLK, CP), lambda i: (i, 0)),
            pl.BlockSpec((K, CP), lambda i: (0, 0)),
            pl.BlockSpec((K, CP), lambda i: (0, 0)),
        ],
        out_specs=pl.BlockSpec(memory_space=pltpu.SMEM),
        out_shape=jax.ShapeDtypeStruct((1, 1), jnp.float32),
        compiler_params=pltpu.CompilerParams(skip_device_barrier=True),
    )(v0, v1, queue_pad, ones_pad)


def kernel(emb0, emb1, perm, queue):
    B, C_, H, W = emb0.shape
    e0 = emb0.reshape(B * C_ * H * W)
    e1 = emb1.reshape(B * C_ * H * W)
    v0, v1 = _sc_gather(e0, e1, perm)
    queue_pad = jnp.pad(queue, ((0, 0), (0, CP - C))).astype(jnp.bfloat16)
    ones_pad = jnp.ones((K, CP), jnp.bfloat16)
    out = _dense_loss(v0.reshape(N, CP), v1.reshape(N, CP), queue_pad, ones_pad)
    return out[0, 0]


# speedup vs baseline: 1.1335x; 1.1335x over previous
import jax
import jax.numpy as jnp
from jax import lax
from jax.experimental import pallas as pl
from jax.experimental.pallas import tpu as pltpu
from jax.experimental.pallas import tpu_sc as plsc

TEMP = 0.07
N = 2048
C = 64
CP = 128
K = 4096
LOG2_HW = 14
LOG2_CHW = 20
NUM_WORKERS = 32
S_PER_W = N // NUM_WORKERS
LANES = 16


def _sc_gather_body(emb0_hbm, emb1_hbm, perm_hbm, out0_hbm, out1_hbm,
                    perm_v, idx_v, rows0_v, rows1_v, sem0, sem1):
    wid = lax.axis_index("s") * 2 + lax.axis_index("c")
    s_base = wid * S_PER_W
    pltpu.sync_copy(perm_hbm.at[pl.ds(s_base, S_PER_W)], perm_v)

    lane = lax.iota(jnp.int32, 16)
    bases = []
    pos0 = []
    for g in range(S_PER_W // LANES):
        p = perm_v[pl.ds(g * LANES, LANES)]
        b = lax.shift_right_logical(p, LOG2_HW)
        hw = lax.bitwise_and(p, (1 << LOG2_HW) - 1)
        bases.append(lax.shift_left(b, LOG2_CHW) + hw)
        pos0.append((g * LANES + lane) * C)

    def build(c, carry):
        cofs = lax.shift_left(c, LOG2_HW)
        for g in range(S_PER_W // LANES):
            pos = pos0[g] + c
            plsc.store_scatter(idx_v, [pos], bases[g] + cofs)
        return carry

    lax.fori_loop(0, C, build, 0)

    def fire(s, carry):
        pltpu.make_async_copy(
            emb0_hbm.at[idx_v.at[pl.ds(s * C, C)]],
            rows0_v.at[pl.ds(s * CP, C)], sem0).start()
        pltpu.make_async_copy(
            emb1_hbm.at[idx_v.at[pl.ds(s * C, C)]],
            rows1_v.at[pl.ds(s * CP, C)], sem1).start()
        return carry

    lax.fori_loop(0, S_PER_W, fire, 0)

    pltpu.make_async_copy(
        emb0_hbm.at[pl.ds(0, S_PER_W * C)],
        rows0_v.at[pl.ds(0, S_PER_W * C)], sem0).wait()
    pltpu.make_async_copy(
        emb1_hbm.at[pl.ds(0, S_PER_W * C)],
        rows1_v.at[pl.ds(0, S_PER_W * C)], sem1).wait()

    pltpu.sync_copy(rows0_v, out0_hbm.at[pl.ds(s_base * CP, S_PER_W * CP)])
    pltpu.sync_copy(rows1_v, out1_hbm.at[pl.ds(s_base * CP, S_PER_W * CP)])


def _sc_gather(emb0_flat, emb1_flat, perm):
    fn = pl.kernel(
        _sc_gather_body,
        mesh=plsc.VectorSubcoreMesh(core_axis_name="c", subcore_axis_name="s"),
        compiler_params=pltpu.CompilerParams(needs_layout_passes=False,
                                             use_tc_tiling_on_sc=False,
                                             skip_device_barrier=True),
        out_type=[jax.ShapeDtypeStruct((N * CP,), jnp.float32),
                  jax.ShapeDtypeStruct((N * CP,), jnp.float32)],
        scratch_types=[
            pltpu.VMEM((S_PER_W,), jnp.int32),
            pltpu.VMEM((S_PER_W * C,), jnp.int32),
            pltpu.VMEM((S_PER_W * CP,), jnp.float32),
            pltpu.VMEM((S_PER_W * CP,), jnp.float32),
            pltpu.SemaphoreType.DMA,
            pltpu.SemaphoreType.DMA,
        ],
    )
    return fn(emb0_flat, emb1_flat, perm)


BLK = 512


def _dense_body(q_ref, k_ref, queue_ref, out_ref):
    i = pl.program_id(0)
    good = lax.broadcasted_iota(jnp.int32, (BLK, CP), 1) < C
    q = jnp.where(good, q_ref[...], 0.0)
    kk = jnp.where(good, k_ref[...], 0.0)
    qn = q / (jnp.sqrt(jnp.sum(q * q, axis=1, keepdims=True)) + 1e-6)
    kn = kk / (jnp.sqrt(jnp.sum(kk * kk, axis=1, keepdims=True)) + 1e-6)
    z = jnp.sum(qn * kn, axis=1, keepdims=True) * (1.0 / TEMP)
    qs = (qn * (1.0 / TEMP)).astype(jnp.bfloat16)
    lneg = lax.dot_general(qs, queue_ref[...], (((1,), (1,)), ((), ())),
                           preferred_element_type=jnp.float32)
    s = jnp.exp(z) + jnp.sum(jnp.exp(lneg), axis=1, keepdims=True)
    lse = jnp.log(s)
    partial = jnp.sum(lse - z) * (1.0 / N)

    @pl.when(i == 0)
    def _():
        out_ref[0, 0] = 0.0

    out_ref[0, 0] += partial


def _dense_loss(v0, v1, queue_pad):
    return pl.pallas_call(
        _dense_body,
        grid=(N // BLK,),
        in_specs=[
            pl.BlockSpec((BLK, CP), lambda i: (i, 0)),
            pl.BlockSpec((BLK, CP), lambda i: (i, 0)),
            pl.BlockSpec((K, CP), lambda i: (0, 0)),
        ],
        out_specs=pl.BlockSpec(memory_space=pltpu.SMEM),
        out_shape=jax.ShapeDtypeStruct((1, 1), jnp.float32),
        compiler_params=pltpu.CompilerParams(skip_device_barrier=True),
    )(v0, v1, queue_pad)


def kernel(emb0, emb1, perm, queue):
    B, C_, H, W = emb0.shape
    e0 = emb0.reshape(B * C_ * H * W)
    e1 = emb1.reshape(B * C_ * H * W)
    v0, v1 = _sc_gather(e0, e1, perm)
    queue_pad = jnp.pad(queue, ((0, 0), (0, CP - C))).astype(jnp.bfloat16)
    out = _dense_loss(v0.reshape(N, CP), v1.reshape(N, CP), queue_pad)
    return out[0, 0]


# speedup vs baseline: 1.1779x; 1.0392x over previous
import jax
import jax.numpy as jnp
from jax import lax
from jax.experimental import pallas as pl
from jax.experimental.pallas import tpu as pltpu
from jax.experimental.pallas import tpu_sc as plsc

TEMP = 0.07
N = 2048
C = 64
CP = 128
K = 4096
LOG2_HW = 14
LOG2_CHW = 20
NUM_WORKERS = 32
S_PER_W = N // NUM_WORKERS
LANES = 16


def _sc_gather_body(emb0_hbm, emb1_hbm, perm_hbm, out0_hbm, out1_hbm,
                    perm_v, idx_v, rows0_v, rows1_v, sem0, sem1):
    wid = lax.axis_index("s") * 2 + lax.axis_index("c")
    s_base = wid * S_PER_W
    pltpu.sync_copy(perm_hbm.at[pl.ds(s_base, S_PER_W)], perm_v)

    lane = lax.iota(jnp.int32, 16)
    for g in range(S_PER_W // LANES):
        p = perm_v[pl.ds(g * LANES, LANES)]
        b = lax.shift_right_logical(p, LOG2_HW)
        hw = lax.bitwise_and(p, (1 << LOG2_HW) - 1)
        base = lax.shift_left(b, LOG2_CHW) + hw
        pos0 = (g * LANES + lane) * C

        def build(c, carry, base=base, pos0=pos0):
            plsc.store_scatter(idx_v, [pos0 + c],
                               base + lax.shift_left(c, LOG2_HW))
            return carry

        lax.fori_loop(0, C, build, 0)

        def fire(s, carry):
            pltpu.make_async_copy(
                emb0_hbm.at[idx_v.at[pl.ds(s * C, C)]],
                rows0_v.at[pl.ds(s * CP, C)], sem0).start()
            pltpu.make_async_copy(
                emb1_hbm.at[idx_v.at[pl.ds(s * C, C)]],
                rows1_v.at[pl.ds(s * CP, C)], sem1).start()
            return carry

        lax.fori_loop(g * LANES, (g + 1) * LANES, fire, 0)

    pltpu.make_async_copy(
        emb0_hbm.at[pl.ds(0, S_PER_W * C)],
        rows0_v.at[pl.ds(0, S_PER_W * C)], sem0).wait()
    pltpu.make_async_copy(
        emb1_hbm.at[pl.ds(0, S_PER_W * C)],
        rows1_v.at[pl.ds(0, S_PER_W * C)], sem1).wait()

    pltpu.sync_copy(rows0_v, out0_hbm.at[pl.ds(s_base * CP, S_PER_W * CP)])
    pltpu.sync_copy(rows1_v, out1_hbm.at[pl.ds(s_base * CP, S_PER_W * CP)])


def _sc_gather(emb0_flat, emb1_flat, perm):
    fn = pl.kernel(
        _sc_gather_body,
        mesh=plsc.VectorSubcoreMesh(core_axis_name="c", subcore_axis_name="s"),
        compiler_params=pltpu.CompilerParams(needs_layout_passes=False,
                                             use_tc_tiling_on_sc=False,
                                             skip_device_barrier=True),
        out_type=[jax.ShapeDtypeStruct((N * CP,), jnp.float32),
                  jax.ShapeDtypeStruct((N * CP,), jnp.float32)],
        scratch_types=[
            pltpu.VMEM((S_PER_W,), jnp.int32),
            pltpu.VMEM((S_PER_W * C,), jnp.int32),
            pltpu.VMEM((S_PER_W * CP,), jnp.float32),
            pltpu.VMEM((S_PER_W * CP,), jnp.float32),
            pltpu.SemaphoreType.DMA,
            pltpu.SemaphoreType.DMA,
        ],
    )
    return fn(emb0_flat, emb1_flat, perm)


BLK = 1024


def _dense_body(q_ref, k_ref, queue_ref, out_ref):
    i = pl.program_id(0)
    good = lax.broadcasted_iota(jnp.int32, (BLK, CP), 1) < C
    q = jnp.where(good, q_ref[...], 0.0)
    kk = jnp.where(good, k_ref[...], 0.0)
    qn = q / (jnp.sqrt(jnp.sum(q * q, axis=1, keepdims=True)) + 1e-6)
    kn = kk / (jnp.sqrt(jnp.sum(kk * kk, axis=1, keepdims=True)) + 1e-6)
    z = jnp.sum(qn * kn, axis=1, keepdims=True) * (1.0 / TEMP)
    qs = (qn * (1.0 / TEMP)).astype(jnp.bfloat16)
    lneg = lax.dot_general(qs, queue_ref[...], (((1,), (1,)), ((), ())),
                           preferred_element_type=jnp.float32)
    s = jnp.exp(z) + jnp.sum(jnp.exp(lneg), axis=1, keepdims=True)
    lse = jnp.log(s)
    partial = jnp.sum(lse - z) * (1.0 / N)

    @pl.when(i == 0)
    def _():
        out_ref[0, 0] = 0.0

    out_ref[0, 0] += partial


def _dense_loss(v0, v1, queue_pad):
    return pl.pallas_call(
        _dense_body,
        grid=(N // BLK,),
        in_specs=[
            pl.BlockSpec((BLK, CP), lambda i: (i, 0)),
            pl.BlockSpec((BLK, CP), lambda i: (i, 0)),
            pl.BlockSpec((K, CP), lambda i: (0, 0)),
        ],
        out_specs=pl.BlockSpec(memory_space=pltpu.SMEM),
        out_shape=jax.ShapeDtypeStruct((1, 1), jnp.float32),
        compiler_params=pltpu.CompilerParams(skip_device_barrier=True),
    )(v0, v1, queue_pad)


def kernel(emb0, emb1, perm, queue):
    B, C_, H, W = emb0.shape
    e0 = emb0.reshape(B * C_ * H * W)
    e1 = emb1.reshape(B * C_ * H * W)
    v0, v1 = _sc_gather(e0, e1, perm)
    queue_pad = jnp.pad(queue, ((0, 0), (0, CP - C))).astype(jnp.bfloat16)
    out = _dense_loss(v0.reshape(N, CP), v1.reshape(N, CP), queue_pad)
    return out[0, 0]


# speedup vs baseline: 1.1818x; 1.0033x over previous
import jax
import jax.numpy as jnp
from jax import lax
from jax.experimental import pallas as pl
from jax.experimental.pallas import tpu as pltpu
from jax.experimental.pallas import tpu_sc as plsc

TEMP = 0.07
N = 2048
C = 64
CP = 128
K = 4096
LOG2_HW = 14
LOG2_CHW = 20
NUM_WORKERS = 32
S_PER_W = N // NUM_WORKERS
LANES = 16


def _sc_gather_body(emb0_hbm, emb1_hbm, perm_hbm, out0_hbm, out1_hbm,
                    perm_v, idx_v, rows0_v, rows1_v, sem0, sem1):
    wid = lax.axis_index("s") * 2 + lax.axis_index("c")
    s_base = wid * S_PER_W
    pltpu.sync_copy(perm_hbm.at[pl.ds(s_base, S_PER_W)], perm_v)

    lane = lax.iota(jnp.int32, 16)

    def group(g, carry):
        p = perm_v[pl.ds(g * LANES, LANES)]
        b = lax.shift_right_logical(p, LOG2_HW)
        hw = lax.bitwise_and(p, (1 << LOG2_HW) - 1)
        base = lax.shift_left(b, LOG2_CHW) + hw
        pos0 = (g * LANES + lane) * C

        def build(c, carry2):
            plsc.store_scatter(idx_v, [pos0 + c],
                               base + lax.shift_left(c, LOG2_HW))
            return carry2

        lax.fori_loop(0, C, build, 0)

        def fire(s, carry2):
            pltpu.make_async_copy(
                emb0_hbm.at[idx_v.at[pl.ds(s * C, C)]],
                rows0_v.at[pl.ds(s * CP, C)], sem0).start()
            pltpu.make_async_copy(
                emb1_hbm.at[idx_v.at[pl.ds(s * C, C)]],
                rows1_v.at[pl.ds(s * CP, C)], sem1).start()
            return carry2

        lax.fori_loop(g * LANES, (g + 1) * LANES, fire, 0)
        return carry

    lax.fori_loop(0, S_PER_W // LANES, group, 0)

    pltpu.make_async_copy(
        emb0_hbm.at[pl.ds(0, S_PER_W * C)],
        rows0_v.at[pl.ds(0, S_PER_W * C)], sem0).wait()
    pltpu.make_async_copy(
        emb1_hbm.at[pl.ds(0, S_PER_W * C)],
        rows1_v.at[pl.ds(0, S_PER_W * C)], sem1).wait()

    pltpu.sync_copy(rows0_v, out0_hbm.at[pl.ds(s_base * CP, S_PER_W * CP)])
    pltpu.sync_copy(rows1_v, out1_hbm.at[pl.ds(s_base * CP, S_PER_W * CP)])


def _sc_gather(emb0_flat, emb1_flat, perm):
    fn = pl.kernel(
        _sc_gather_body,
        mesh=plsc.VectorSubcoreMesh(core_axis_name="c", subcore_axis_name="s"),
        compiler_params=pltpu.CompilerParams(needs_layout_passes=False,
                                             use_tc_tiling_on_sc=False,
                                             skip_device_barrier=True),
        out_type=[jax.ShapeDtypeStruct((N * CP,), jnp.float32),
                  jax.ShapeDtypeStruct((N * CP,), jnp.float32)],
        scratch_types=[
            pltpu.VMEM((S_PER_W,), jnp.int32),
            pltpu.VMEM((S_PER_W * C,), jnp.int32),
            pltpu.VMEM((S_PER_W * CP,), jnp.float32),
            pltpu.VMEM((S_PER_W * CP,), jnp.float32),
            pltpu.SemaphoreType.DMA,
            pltpu.SemaphoreType.DMA,
        ],
    )
    return fn(emb0_flat, emb1_flat, perm)


BLK = 1024


def _dense_body(q_ref, k_ref, queue_ref, out_ref):
    i = pl.program_id(0)
    good = lax.broadcasted_iota(jnp.int32, (BLK, CP), 1) < C
    q = jnp.where(good, q_ref[...], 0.0)
    kk = jnp.where(good, k_ref[...], 0.0)
    qn = q / (jnp.sqrt(jnp.sum(q * q, axis=1, keepdims=True)) + 1e-6)
    kn = kk / (jnp.sqrt(jnp.sum(kk * kk, axis=1, keepdims=True)) + 1e-6)
    z = jnp.sum(qn * kn, axis=1, keepdims=True) * (1.0 / TEMP)
    qs = (qn * (1.0 / TEMP)).astype(jnp.bfloat16)
    lneg = lax.dot_general(qs, queue_ref[...], (((1,), (1,)), ((), ())),
                           preferred_element_type=jnp.float32)
    s = jnp.exp(z) + jnp.sum(jnp.exp(lneg), axis=1, keepdims=True)
    lse = jnp.log(s)
    partial = jnp.sum(lse - z) * (1.0 / N)

    @pl.when(i == 0)
    def _():
        out_ref[0, 0] = 0.0

    out_ref[0, 0] += partial


def _dense_loss(v0, v1, queue_pad):
    return pl.pallas_call(
        _dense_body,
        grid=(N // BLK,),
        in_specs=[
            pl.BlockSpec((BLK, CP), lambda i: (i, 0)),
            pl.BlockSpec((BLK, CP), lambda i: (i, 0)),
            pl.BlockSpec((K, CP), lambda i: (0, 0)),
        ],
        out_specs=pl.BlockSpec(memory_space=pltpu.SMEM),
        out_shape=jax.ShapeDtypeStruct((1, 1), jnp.float32),
        compiler_params=pltpu.CompilerParams(skip_device_barrier=True),
    )(v0, v1, queue_pad)


def kernel(emb0, emb1, perm, queue):
    B, C_, H, W = emb0.shape
    e0 = emb0.reshape(B * C_ * H * W)
    e1 = emb1.reshape(B * C_ * H * W)
    v0, v1 = _sc_gather(e0, e1, perm)
    queue_pad = jnp.pad(queue, ((0, 0), (0, CP - C))).astype(jnp.bfloat16)
    out = _dense_loss(v0.reshape(N, CP), v1.reshape(N, CP), queue_pad)
    return out[0, 0]


# speedup vs baseline: 1.1929x; 1.0094x over previous
import jax
import jax.numpy as jnp
from jax import lax
from jax.experimental import pallas as pl
from jax.experimental.pallas import tpu as pltpu
from jax.experimental.pallas import tpu_sc as plsc

TEMP = 0.07
N = 2048
C = 64
CP = 128
K = 4096
LOG2_HW = 14
LOG2_CHW = 20
NUM_WORKERS = 32
S_PER_W = N // NUM_WORKERS
LANES = 16


def _sc_gather_body(emb0_hbm, emb1_hbm, perm_hbm, out0_hbm, out1_hbm,
                    perm_v, idx_v, rows0_v, rows1_v, sem0, sem1):
    wid = lax.axis_index("s") * 2 + lax.axis_index("c")
    s_base = wid * S_PER_W
    pltpu.sync_copy(perm_hbm.at[pl.ds(s_base, S_PER_W)], perm_v)

    lane = lax.iota(jnp.int32, 16)

    def group(g, carry):
        p = perm_v[pl.ds(g * LANES, LANES)]
        b = lax.shift_right_logical(p, LOG2_HW)
        hw = lax.bitwise_and(p, (1 << LOG2_HW) - 1)
        base = lax.shift_left(b, LOG2_CHW) + hw
        pos0 = (g * LANES + lane) * C

        def build(c, carry2):
            plsc.store_scatter(idx_v, [pos0 + c],
                               base + lax.shift_left(c, LOG2_HW))
            return carry2

        lax.fori_loop(0, C, build, 0)

        def fire(s, carry2):
            pltpu.make_async_copy(
                emb0_hbm.at[idx_v.at[pl.ds(s * C, C)]],
                rows0_v.at[pl.ds(s * CP, C)], sem0).start()
            pltpu.make_async_copy(
                emb1_hbm.at[idx_v.at[pl.ds(s * C, C)]],
                rows1_v.at[pl.ds(s * CP, C)], sem1).start()
            return carry2

        lax.fori_loop(g * LANES, (g + 1) * LANES, fire, 0)
        return carry

    lax.fori_loop(0, S_PER_W // LANES, group, 0)

    pltpu.make_async_copy(
        emb0_hbm.at[pl.ds(0, S_PER_W * C)],
        rows0_v.at[pl.ds(0, S_PER_W * C)], sem0).wait()
    pltpu.make_async_copy(
        emb1_hbm.at[pl.ds(0, S_PER_W * C)],
        rows1_v.at[pl.ds(0, S_PER_W * C)], sem1).wait()

    pltpu.sync_copy(rows0_v, out0_hbm.at[pl.ds(s_base * CP, S_PER_W * CP)])
    pltpu.sync_copy(rows1_v, out1_hbm.at[pl.ds(s_base * CP, S_PER_W * CP)])


def _sc_gather(emb0_flat, emb1_flat, perm):
    fn = pl.kernel(
        _sc_gather_body,
        mesh=plsc.VectorSubcoreMesh(core_axis_name="c", subcore_axis_name="s"),
        compiler_params=pltpu.CompilerParams(needs_layout_passes=False,
                                             use_tc_tiling_on_sc=False,
                                             skip_device_barrier=True),
        out_type=[jax.ShapeDtypeStruct((N * CP,), jnp.float32),
                  jax.ShapeDtypeStruct((N * CP,), jnp.float32)],
        scratch_types=[
            pltpu.VMEM((S_PER_W,), jnp.int32),
            pltpu.VMEM((S_PER_W * C,), jnp.int32),
            pltpu.VMEM((S_PER_W * CP,), jnp.float32),
            pltpu.VMEM((S_PER_W * CP,), jnp.float32),
            pltpu.SemaphoreType.DMA,
            pltpu.SemaphoreType.DMA,
        ],
    )
    return fn(emb0_flat, emb1_flat, perm)


BLK = 2048


def _dense_body(q_ref, k_ref, queue_ref, out_ref):
    i = pl.program_id(0)
    good = lax.broadcasted_iota(jnp.int32, (BLK, CP), 1) < C
    q = jnp.where(good, q_ref[...], 0.0)
    kk = jnp.where(good, k_ref[...], 0.0)
    qn = q / (jnp.sqrt(jnp.sum(q * q, axis=1, keepdims=True)) + 1e-6)
    kn = kk / (jnp.sqrt(jnp.sum(kk * kk, axis=1, keepdims=True)) + 1e-6)
    z = jnp.sum(qn * kn, axis=1, keepdims=True) * (1.0 / TEMP)
    qs = (qn * (1.0 / TEMP)).astype(jnp.bfloat16)
    lneg = lax.dot_general(qs, queue_ref[...], (((1,), (1,)), ((), ())),
                           preferred_element_type=jnp.float32)
    s = jnp.exp(z) + jnp.sum(jnp.exp(lneg), axis=1, keepdims=True)
    lse = jnp.log(s)
    partial = jnp.sum(lse - z) * (1.0 / N)

    @pl.when(i == 0)
    def _():
        out_ref[0, 0] = 0.0

    out_ref[0, 0] += partial


def _dense_loss(v0, v1, queue_pad):
    return pl.pallas_call(
        _dense_body,
        grid=(N // BLK,),
        in_specs=[
            pl.BlockSpec((BLK, CP), lambda i: (i, 0)),
            pl.BlockSpec((BLK, CP), lambda i: (i, 0)),
            pl.BlockSpec((K, CP), lambda i: (0, 0)),
        ],
        out_specs=pl.BlockSpec(memory_space=pltpu.SMEM),
        out_shape=jax.ShapeDtypeStruct((1, 1), jnp.float32),
        compiler_params=pltpu.CompilerParams(skip_device_barrier=True),
    )(v0, v1, queue_pad)


def kernel(emb0, emb1, perm, queue):
    B, C_, H, W = emb0.shape
    e0 = emb0.reshape(B * C_ * H * W)
    e1 = emb1.reshape(B * C_ * H * W)
    v0, v1 = _sc_gather(e0, e1, perm)
    queue_pad = jnp.pad(queue, ((0, 0), (0, CP - C))).astype(jnp.bfloat16)
    out = _dense_loss(v0.reshape(N, CP), v1.reshape(N, CP), queue_pad)
    return out[0, 0]


# speedup vs baseline: 1.2128x; 1.0167x over previous
import jax
import jax.numpy as jnp
from jax import lax
from jax.experimental import pallas as pl
from jax.experimental.pallas import tpu as pltpu
from jax.experimental.pallas import tpu_sc as plsc

TEMP = 0.07
N = 2048
C = 64
CP = 128
K = 4096
LOG2_HW = 14
LOG2_CHW = 20
NUM_WORKERS = 32
S_PER_W = N // NUM_WORKERS
LANES = 16


def _sc_gather_body(emb0_hbm, emb1_hbm, perm_hbm, out_hbm,
                    perm_v, idx_v, rows_v, sem0, sem1):
    wid = lax.axis_index("s") * 2 + lax.axis_index("c")
    s_base = wid * S_PER_W
    pltpu.sync_copy(perm_hbm.at[pl.ds(s_base, S_PER_W)], perm_v)

    lane = lax.iota(jnp.int32, 16)

    def group(g, carry):
        p = perm_v[pl.ds(g * LANES, LANES)]
        b = lax.shift_right_logical(p, LOG2_HW)
        hw = lax.bitwise_and(p, (1 << LOG2_HW) - 1)
        base = lax.shift_left(b, LOG2_CHW) + hw
        pos0 = (g * LANES + lane) * C

        def build(c, carry2):
            plsc.store_scatter(idx_v, [pos0 + c],
                               base + lax.shift_left(c, LOG2_HW))
            return carry2

        lax.fori_loop(0, C, build, 0)

        def fire(s, carry2):
            pltpu.make_async_copy(
                emb0_hbm.at[idx_v.at[pl.ds(s * C, C)]],
                rows_v.at[pl.ds(s * CP, C)], sem0).start()
            pltpu.make_async_copy(
                emb1_hbm.at[idx_v.at[pl.ds(s * C, C)]],
                rows_v.at[pl.ds(s * CP + C, C)], sem1).start()
            return carry2

        lax.fori_loop(g * LANES, (g + 1) * LANES, fire, 0)
        return carry

    lax.fori_loop(0, S_PER_W // LANES, group, 0)

    pltpu.make_async_copy(
        emb0_hbm.at[pl.ds(0, S_PER_W * C)],
        rows_v.at[pl.ds(0, S_PER_W * C)], sem0).wait()
    pltpu.make_async_copy(
        emb1_hbm.at[pl.ds(0, S_PER_W * C)],
        rows_v.at[pl.ds(0, S_PER_W * C)], sem1).wait()

    pltpu.sync_copy(rows_v, out_hbm.at[pl.ds(s_base * CP, S_PER_W * CP)])


def _sc_gather(emb0_flat, emb1_flat, perm):
    fn = pl.kernel(
        _sc_gather_body,
        mesh=plsc.VectorSubcoreMesh(core_axis_name="c", subcore_axis_name="s"),
        compiler_params=pltpu.CompilerParams(needs_layout_passes=False,
                                             use_tc_tiling_on_sc=False,
                                             skip_device_barrier=True),
        out_type=jax.ShapeDtypeStruct((N * CP,), jnp.float32),
        scratch_types=[
            pltpu.VMEM((S_PER_W,), jnp.int32),
            pltpu.VMEM((S_PER_W * C,), jnp.int32),
            pltpu.VMEM((S_PER_W * CP,), jnp.float32),
            pltpu.SemaphoreType.DMA,
            pltpu.SemaphoreType.DMA,
        ],
    )
    return fn(emb0_flat, emb1_flat, perm)


BLK = 2048


def _dense_body(x_ref, queue_ref, out_ref):
    i = pl.program_id(0)
    good = lax.broadcasted_iota(jnp.int32, (BLK, CP), 1) < C
    x = x_ref[...]
    xr = jnp.concatenate([x[:, C:], x[:, :C]], axis=1)
    q = jnp.where(good, x, 0.0)
    kk = jnp.where(good, xr, 0.0)
    qn = q / (jnp.sqrt(jnp.sum(q * q, axis=1, keepdims=True)) + 1e-6)
    kn = kk / (jnp.sqrt(jnp.sum(kk * kk, axis=1, keepdims=True)) + 1e-6)
    z = jnp.sum(qn * kn, axis=1, keepdims=True) * (1.0 / TEMP)
    qs = (qn * (1.0 / TEMP)).astype(jnp.bfloat16)
    lneg = lax.dot_general(qs, queue_ref[...], (((1,), (1,)), ((), ())),
                           preferred_element_type=jnp.float32)
    s = jnp.exp(z) + jnp.sum(jnp.exp(lneg), axis=1, keepdims=True)
    lse = jnp.log(s)
    partial = jnp.sum(lse - z) * (1.0 / N)

    @pl.when(i == 0)
    def _():
        out_ref[0, 0] = 0.0

    out_ref[0, 0] += partial


def _dense_loss(v, queue_pad):
    return pl.pallas_call(
        _dense_body,
        grid=(N // BLK,),
        in_specs=[
            pl.BlockSpec((BLK, CP), lambda i: (i, 0)),
            pl.BlockSpec((K, CP), lambda i: (0, 0)),
        ],
        out_specs=pl.BlockSpec(memory_space=pltpu.SMEM),
        out_shape=jax.ShapeDtypeStruct((1, 1), jnp.float32),
        compiler_params=pltpu.CompilerParams(skip_device_barrier=True),
    )(v, queue_pad)


def kernel(emb0, emb1, perm, queue):
    B, C_, H, W = emb0.shape
    e0 = emb0.reshape(B * C_ * H * W)
    e1 = emb1.reshape(B * C_ * H * W)
    v = _sc_gather(e0, e1, perm)
    queue_pad = jnp.pad(queue, ((0, 0), (0, CP - C))).astype(jnp.bfloat16)
    out = _dense_loss(v.reshape(N, CP), queue_pad)
    return out[0, 0]
